# Initial kernel scaffold; baseline (speedup 1.0000x reference)
import jax
import jax.numpy as jnp
from jax.experimental import pallas as pl


def kernel(x, x_mask, x_idx, keys_store, store_vals, neighbor_feats, W_enc, b_enc, W_cls, b_cls, W1, b1, W2, b2):
    raise NotImplementedError("write your pallas kernel here")


# speedup vs baseline: 6.1138x; 6.1138x over previous
import functools

import jax
import jax.numpy as jnp
from jax import lax
from jax.experimental import pallas as pl
from jax.experimental.pallas import tpu as pltpu

B = 128
S = 128
D = 768
K_STORE = 100000
NUM_CLASSES = 1000
K = 32
TEMP = 10.0

K_PAD = 102400
BLK = 128
NBLK = K_PAD // BLK
CHUNK = 4096
NCHUNK = K_PAD // CHUNK
BPC = CHUNK // BLK

_NEG = -1e30


def _enc_body(x_ref, m_ref, wenc_ref, benc_ref, wcls_ref, bcls_ref,
              trep_ref, mprob_ref, acc_ref, msum_ref):
    i = pl.program_id(0)

    @pl.when(i == 0)
    def _():
        acc_ref[...] = jnp.zeros_like(acc_ref)
        msum_ref[...] = jnp.zeros_like(msum_ref)

    xb = x_ref[...]
    mb = m_ref[0]
    acc_ref[...] += jnp.sum(xb * mb[:, :, None], axis=1)
    msum_ref[...] += jnp.sum(mb, axis=1, keepdims=True)

    @pl.when(i == pl.num_programs(0) - 1)
    def _():
        pooled = acc_ref[...] / jnp.maximum(msum_ref[...], 1.0)
        t = jnp.tanh(
            jnp.dot(pooled, wenc_ref[...],
                    preferred_element_type=jnp.float32) + benc_ref[...])
        trep_ref[...] = t
        logits = jnp.dot(t, wcls_ref[...],
                         preferred_element_type=jnp.float32) + bcls_ref[...]
        m = jnp.max(logits, axis=1, keepdims=True)
        e = jnp.exp(logits - m)
        mprob_ref[...] = e / jnp.sum(e, axis=1, keepdims=True)


def _encode(x, x_mask, W_enc, b_enc, W_cls, b_cls):
    nsteps = 16
    sblk = S // nsteps
    fn = pl.pallas_call(
        _enc_body,
        grid=(nsteps,),
        in_specs=[
            pl.BlockSpec((B, sblk, D), lambda i: (0, i, 0)),
            pl.BlockSpec((1, B, sblk), lambda i: (i, 0, 0)),
            pl.BlockSpec((D, D), lambda i: (0, 0)),
            pl.BlockSpec((D,), lambda i: (0,)),
            pl.BlockSpec((D, NUM_CLASSES), lambda i: (0, 0)),
            pl.BlockSpec((NUM_CLASSES,), lambda i: (0,)),
        ],
        out_specs=[
            pl.BlockSpec((B, D), lambda i: (0, 0)),
            pl.BlockSpec((B, NUM_CLASSES), lambda i: (0, 0)),
        ],
        out_shape=[
            jax.ShapeDtypeStruct((B, D), jnp.float32),
            jax.ShapeDtypeStruct((B, NUM_CLASSES), jnp.float32),
        ],
        scratch_shapes=[
            pltpu.VMEM((B, D), jnp.float32),
            pltpu.VMEM((B, 1), jnp.float32),
        ],
    )
    m3 = x_mask.reshape(B, nsteps, sblk).transpose(1, 0, 2)
    return fn(x, m3, W_enc, b_enc, W_cls, b_cls)


def _score_body(q_ref, xidx_ref, keys_ref, s_ref, bm3_ref, tau_ref,
                tacc_ref):
    c = pl.program_id(0)

    @pl.when(c == 0)
    def _():
        tacc_ref[...] = jnp.full_like(tacc_ref, _NEG)

    q = q_ref[...]
    kc = keys_ref[...]
    q2 = jnp.sum(q * q, axis=1, keepdims=True)
    k2 = jnp.sum(kc * kc, axis=1)
    qk = lax.dot_general(q, kc, (((1,), (1,)), ((), ())),
                         preferred_element_type=jnp.float32)
    s = -(q2 + k2[None, :] - 2.0 * qk)
    cols = c * CHUNK + lax.broadcasted_iota(jnp.int32, (B, CHUNK), 1)
    s = jnp.where((cols == xidx_ref[...]) | (cols >= K_STORE), _NEG, s)
    s_ref[...] = s
    maxes = jnp.max(s.reshape(B, BPC, BLK), axis=2)
    bm3_ref[0] = maxes
    cmin = jnp.min(maxes, axis=1, keepdims=True)
    tacc_ref[...] = jnp.maximum(tacc_ref[...], cmin)

    @pl.when(c == pl.num_programs(0) - 1)
    def _():
        tau_ref[...] = jnp.broadcast_to(tacc_ref[...], (B, 128))


def _scores(q, x_idx, keys_store):
    return pl.pallas_call(
        _score_body,
        grid=(NCHUNK,),
        in_specs=[
            pl.BlockSpec((B, D), lambda c: (0, 0)),
            pl.BlockSpec((B, 1), lambda c: (0, 0)),
            pl.BlockSpec((CHUNK, D), lambda c: (c, 0)),
        ],
        out_specs=[
            pl.BlockSpec((B, CHUNK), lambda c: (0, c)),
            pl.BlockSpec((1, B, BPC), lambda c: (c, 0, 0)),
            pl.BlockSpec((B, 128), lambda c: (0, 0)),
        ],
        out_shape=[
            jax.ShapeDtypeStruct((B, K_PAD), jnp.float32),
            jax.ShapeDtypeStruct((NCHUNK, B, BPC), jnp.float32),
            jax.ShapeDtypeStruct((B, 128), jnp.float32),
        ],
        scratch_shapes=[pltpu.VMEM((B, 1), jnp.float32)],
    )(q, x_idx.reshape(B, 1), keys_store)


def _comb_body(nbr_ref, wenc_ref, benc_ref, probs_ref, trep_ref,
               kprob_ref, mprob_ref, w1_ref, b1_ref, w2_ref, b2_ref,
               out_ref):
    nbr = nbr_ref[...]
    h = jnp.tanh(jnp.dot(nbr, wenc_ref[...],
                         preferred_element_type=jnp.float32) + benc_ref[...])
    h3 = h.reshape(B, K, D)
    probs = probs_ref[...]
    nrep = jnp.sum(probs[:, :, None] * h3, axis=1)
    t = trep_ref[...]
    hh = (jnp.dot(t, w1_ref[0:D, :], preferred_element_type=jnp.float32)
          + jnp.dot(nrep, w1_ref[D:2 * D, :],
                    preferred_element_type=jnp.float32) + b1_ref[...])
    g = jnp.dot(hh, w2_ref[...], preferred_element_type=jnp.float32) \
        + b2_ref[...]
    p = jax.nn.sigmoid(g)
    out_ref[...] = jnp.log(p * kprob_ref[...] + (1.0 - p) * mprob_ref[...]
                           + 1e-12)


def _combine(nbr2d, W_enc, b_enc, probs, text_rep, knn_prob, model_prob,
             W1, b1, W2, b2):
    return pl.pallas_call(
        _comb_body,
        out_shape=jax.ShapeDtypeStruct((B, NUM_CLASSES), jnp.float32),
    )(nbr2d, W_enc, b_enc, probs, text_rep, knn_prob, model_prob,
      W1, b1, W2, b2)


def kernel(x, x_mask, x_idx, keys_store, store_vals, neighbor_feats,
           W_enc, b_enc, W_cls, b_cls, W1, b1, W2, b2):
    x_idx = x_idx.astype(jnp.int32)
    text_rep, model_prob = _encode(x, x_mask, W_enc, b_enc, W_cls, b_cls)

    q = keys_store[x_idx]
    scores, bmax3, tau = _scores(q, x_idx, keys_store)

    _, knns = lax.top_k(scores, K)
    labels = store_vals[knns]
    knn_keys = keys_store[knns]
    dists = jnp.sum((text_rep[:, None, :] - knn_keys) ** 2, axis=-1)
    probs = jax.nn.softmax(-dists / TEMP, axis=-1)
    rows = jnp.broadcast_to(jnp.arange(B)[:, None], (B, K))
    knn_prob = jnp.zeros((B, NUM_CLASSES), jnp.float32).at[rows, labels].add(probs)
    nbr2d = neighbor_feats[knns].reshape(B * K, D)

    return _combine(nbr2d, W_enc, b_enc, probs, text_rep, knn_prob,
                    model_prob, W1, b1, W2, b2)


# speedup vs baseline: 10.3222x; 1.6883x over previous
import functools

import jax
import jax.numpy as jnp
from jax import lax
from jax.experimental import pallas as pl
from jax.experimental.pallas import tpu as pltpu
from jax.experimental.pallas import tpu_sc as plsc

B = 128
S = 128
D = 768
K_STORE = 100000
NUM_CLASSES = 1000
K = 32
TEMP = 10.0

K_PAD = 102400
BLK = 128
NBLK = K_PAD // BLK
CHUNK = 4096
NCHUNK = K_PAD // CHUNK
BPC = CHUNK // BLK

_NEG = -1e30


def _enc_body(x_ref, m_ref, wenc_ref, benc_ref, wcls_ref, bcls_ref,
              trep_ref, mprob_ref, acc_ref, msum_ref):
    i = pl.program_id(0)

    @pl.when(i == 0)
    def _():
        acc_ref[...] = jnp.zeros_like(acc_ref)
        msum_ref[...] = jnp.zeros_like(msum_ref)

    xb = x_ref[...]
    mb = m_ref[0]
    acc_ref[...] += jnp.sum(xb * mb[:, :, None], axis=1)
    msum_ref[...] += jnp.sum(mb, axis=1, keepdims=True)

    @pl.when(i == pl.num_programs(0) - 1)
    def _():
        pooled = acc_ref[...] / jnp.maximum(msum_ref[...], 1.0)
        t = jnp.tanh(
            jnp.dot(pooled, wenc_ref[...],
                    preferred_element_type=jnp.float32) + benc_ref[...])
        trep_ref[...] = t
        logits = jnp.dot(t, wcls_ref[...],
                         preferred_element_type=jnp.float32) + bcls_ref[...]
        m = jnp.max(logits, axis=1, keepdims=True)
        e = jnp.exp(logits - m)
        mprob_ref[...] = e / jnp.sum(e, axis=1, keepdims=True)


def _encode(x, x_mask, W_enc, b_enc, W_cls, b_cls):
    nsteps = 16
    sblk = S // nsteps
    fn = pl.pallas_call(
        _enc_body,
        grid=(nsteps,),
        in_specs=[
            pl.BlockSpec((B, sblk, D), lambda i: (0, i, 0)),
            pl.BlockSpec((1, B, sblk), lambda i: (i, 0, 0)),
            pl.BlockSpec((D, D), lambda i: (0, 0)),
            pl.BlockSpec((D,), lambda i: (0,)),
            pl.BlockSpec((D, NUM_CLASSES), lambda i: (0, 0)),
            pl.BlockSpec((NUM_CLASSES,), lambda i: (0,)),
        ],
        out_specs=[
            pl.BlockSpec((B, D), lambda i: (0, 0)),
            pl.BlockSpec((B, NUM_CLASSES), lambda i: (0, 0)),
        ],
        out_shape=[
            jax.ShapeDtypeStruct((B, D), jnp.float32),
            jax.ShapeDtypeStruct((B, NUM_CLASSES), jnp.float32),
        ],
        scratch_shapes=[
            pltpu.VMEM((B, D), jnp.float32),
            pltpu.VMEM((B, 1), jnp.float32),
        ],
    )
    m3 = x_mask.reshape(B, nsteps, sblk).transpose(1, 0, 2)
    return fn(x, m3, W_enc, b_enc, W_cls, b_cls)


def _score_body(q_ref, xidx_ref, keys_ref, s_ref, bm3_ref):
    c = pl.program_id(0)
    q = q_ref[...]
    kc = keys_ref[...]
    q2 = jnp.sum(q * q, axis=1, keepdims=True)
    k2 = jnp.sum(kc * kc, axis=1)
    qk = lax.dot_general(q, kc, (((1,), (1,)), ((), ())),
                         preferred_element_type=jnp.float32)
    s = -(q2 + k2[None, :] - 2.0 * qk)
    cols = c * CHUNK + lax.broadcasted_iota(jnp.int32, (B, CHUNK), 1)
    s = jnp.where((cols == xidx_ref[...]) | (cols >= K_STORE), _NEG, s)
    s_ref[...] = s
    bm3_ref[0] = jnp.max(s.reshape(B, BPC, BLK), axis=2)


def _scores(q, x_idx, keys_store):
    return pl.pallas_call(
        _score_body,
        grid=(NCHUNK,),
        in_specs=[
            pl.BlockSpec((B, D), lambda c: (0, 0)),
            pl.BlockSpec((B, 1), lambda c: (0, 0)),
            pl.BlockSpec((CHUNK, D), lambda c: (c, 0)),
        ],
        out_specs=[
            pl.BlockSpec((B, CHUNK), lambda c: (0, c)),
            pl.BlockSpec((1, B, BPC), lambda c: (c, 0, 0)),
        ],
        out_shape=[
            jax.ShapeDtypeStruct((B, K_PAD), jnp.float32),
            jax.ShapeDtypeStruct((NCHUNK, B, BPC), jnp.float32),
        ],
    )(q, x_idx.reshape(B, 1), keys_store)


def _tau_body(bm_ref, tau_ref):
    def step(_, carry):
        bm, m = carry
        m = jnp.max(bm, axis=1, keepdims=True)
        bm = jnp.where(bm == m, _NEG, bm)
        return bm, m
    _, tau = lax.fori_loop(0, K, step,
                           (bm_ref[...], jnp.zeros((B, 1), jnp.float32)))
    tau_ref[...] = jnp.broadcast_to(tau, (B, 16))


def _tau(bm2):
    return pl.pallas_call(
        _tau_body,
        out_shape=jax.ShapeDtypeStruct((B, 16), jnp.float32),
    )(bm2)


NW = 32
RP = B // NW
NHIT = 128
NCAND = 512
NBV = NBLK // 16


def _sc_select():
    mesh = plsc.VectorSubcoreMesh(core_axis_name="c", subcore_axis_name="s")

    @functools.partial(
        pl.kernel,
        out_type=[
            jax.ShapeDtypeStruct((B, K, D), jnp.float32),
            jax.ShapeDtypeStruct((B, K, D), jnp.float32),
            jax.ShapeDtypeStruct((B, K), jnp.int32),
        ],
        mesh=mesh,
        compiler_params=pltpu.CompilerParams(needs_layout_passes=False),
        scratch_types=[
            pltpu.VMEM((NBLK,), jnp.float32),
            pltpu.VMEM((16,), jnp.float32),
            pltpu.VMEM((NHIT,), jnp.int32),
            pltpu.VMEM((NHIT,), jnp.int32),
            pltpu.VMEM((NHIT, BLK), jnp.float32),
            pltpu.VMEM((NCAND,), jnp.float32),
            pltpu.VMEM((NCAND,), jnp.int32),
            pltpu.VMEM((K,), jnp.int32),
            pltpu.VMEM((K, D), jnp.float32),
            pltpu.VMEM((K, D), jnp.float32),
            pltpu.VMEM((K,), jnp.int32),
            pltpu.SemaphoreType.DMA,
            pltpu.SemaphoreType.DMA,
            pltpu.SemaphoreType.DMA,
        ],
    )
    def body(scores2d, bm2, tau, keys_hbm, nbrf_hbm, svals_hbm,
             kk_out, nbr_out, lab_out,
             bm_t, tau_t, hid_t, base_t, blocks_t, cv_t, ci_t, kn_t,
             keyr_t, nbrr_t, lab_t, sem1, sem2, sem3):
        wid = lax.axis_index("s") * 2 + lax.axis_index("c")
        lanes = lax.iota(jnp.int32, 16)
        neg = jnp.full((16,), _NEG, jnp.float32)

        def row_body(r, _unused):
            b = wid * RP + r
            pltpu.sync_copy(bm2.at[b], bm_t)
            pltpu.sync_copy(tau.at[b], tau_t)
            tv = tau_t[...]

            for i in range(NHIT // 16):
                base_t[pl.ds(i * 16, 16)] = jnp.zeros((16,), jnp.int32)
            for i in range(NCAND // 16):
                cv_t[pl.ds(i * 16, 16)] = neg

            def scan_body(i, cnt):
                v = bm_t[pl.ds(i * 16, 16)]
                m = v >= tv
                ids = i * 16 + lanes
                plsc.store_compressed(hid_t.at[pl.ds(cnt, 16)], ids, mask=m)
                plsc.store_compressed(base_t.at[pl.ds(cnt, 16)],
                                      b * NBLK + ids, mask=m)
                c16 = plsc.all_reduce_population_count(m)
                return jnp.minimum(cnt + jnp.max(c16), NHIT - 16)
            cnt = lax.fori_loop(0, NBV, scan_body, 0)

            pltpu.async_copy(scores2d.at[base_t], blocks_t, sem1).wait()

            def cand_body(j, cc):
                hid = plsc.load_gather(hid_t, [jnp.full((16,), j, jnp.int32)])
                cc2 = cc
                for i in range(BLK // 16):
                    v = blocks_t[j, pl.ds(i * 16, 16)]
                    m = v >= tv
                    gcol = hid * BLK + i * 16 + lanes
                    plsc.store_compressed(cv_t.at[pl.ds(cc2, 16)], v, mask=m)
                    plsc.store_compressed(ci_t.at[pl.ds(cc2, 16)], gcol, mask=m)
                    c16 = plsc.all_reduce_population_count(m)
                    cc2 = jnp.minimum(cc2 + jnp.max(c16), NCAND - 16)
                return cc2
            ncand = lax.fori_loop(0, cnt, cand_body, 0)
            nv = (ncand + 15) // 16

            def topk_body(k, carry):
                kn0, kn1 = carry

                def mx_body(i, m):
                    return jnp.maximum(m, cv_t[pl.ds(i * 16, 16)])
                m = lax.fori_loop(0, nv, mx_body, neg)
                mval = jnp.max(m)

                def sel_body(i, acc):
                    v = cv_t[pl.ds(i * 16, 16)]
                    eq = v == mval
                    idxs = ci_t[pl.ds(i * 16, 16)]
                    sel = jnp.max(jnp.where(eq, idxs,
                                            jnp.full((16,), -1, jnp.int32)))
                    cv_t[pl.ds(i * 16, 16)] = jnp.where(eq, neg, v)
                    return jnp.maximum(acc, sel)
                sel = lax.fori_loop(0, nv, sel_body, -1)
                sel = jnp.maximum(sel, 0)
                kn0 = jnp.where((k < 16) & (lanes == k), sel, kn0)
                kn1 = jnp.where((k >= 16) & (lanes == (k - 16)), sel, kn1)
                return kn0, kn1
            z16 = jnp.zeros((16,), jnp.int32)
            kn0, kn1 = lax.fori_loop(0, K, topk_body, (z16, z16))
            kn_t[pl.ds(0, 16)] = kn0
            kn_t[pl.ds(16, 16)] = kn1

            cp1 = pltpu.async_copy(keys_hbm.at[kn_t], keyr_t, sem1)
            cp2 = pltpu.async_copy(nbrf_hbm.at[kn_t], nbrr_t, sem2)
            cp3 = pltpu.async_copy(svals_hbm.at[kn_t], lab_t, sem3)
            cp1.wait()
            cp2.wait()
            cp3.wait()
            pltpu.sync_copy(keyr_t, kk_out.at[b])
            pltpu.sync_copy(nbrr_t, nbr_out.at[b])
            pltpu.sync_copy(lab_t, lab_out.at[b])
            return 0

        lax.fori_loop(0, RP, row_body, 0)

    return body


def _sc_qgather():
    mesh = plsc.VectorSubcoreMesh(core_axis_name="c", subcore_axis_name="s")

    @functools.partial(
        pl.kernel,
        out_type=jax.ShapeDtypeStruct((B, D), jnp.float32),
        mesh=mesh,
        compiler_params=pltpu.CompilerParams(needs_layout_passes=False),
        scratch_types=[
            pltpu.VMEM((8,), jnp.int32),
            pltpu.VMEM((8, D), jnp.float32),
            pltpu.SemaphoreType.DMA,
        ],
    )
    def body(xidx_hbm, keys_hbm, q_out, idx_t, rows_t, sem):
        wid = lax.axis_index("s") * 2 + lax.axis_index("c")

        @pl.when(wid < 16)
        def _():
            base = wid * 8
            pltpu.sync_copy(xidx_hbm.at[pl.ds(base, 8)], idx_t)
            pltpu.async_copy(keys_hbm.at[idx_t], rows_t, sem).wait()
            pltpu.sync_copy(rows_t, q_out.at[pl.ds(base, 8)])

    return body


GB = 32
NG = B // GB


def _comb_body(kk_ref, nbr_ref, lab_ref, wenc_ref, benc_ref, trep_ref,
               mprob_ref, w1_ref, b1_ref, w2_ref, b2_ref, out_ref):
    t = trep_ref[...]
    kk = kk_ref[...]
    dists = jnp.sum((t[:, None, :] - kk) ** 2, axis=-1)
    logit = -dists / TEMP
    mx = jnp.max(logit, axis=1, keepdims=True)
    e = jnp.exp(logit - mx)
    probs = e / jnp.sum(e, axis=1, keepdims=True)

    lab = lab_ref[...]
    cls = lax.broadcasted_iota(jnp.int32, (GB, NUM_CLASSES), 1)
    kprob = jnp.zeros((GB, NUM_CLASSES), jnp.float32)
    for k in range(K):
        kprob = kprob + jnp.where(lab[:, k:k + 1] == cls,
                                  probs[:, k:k + 1], 0.0)

    nbr = nbr_ref[...]
    h = jnp.tanh(jnp.dot(nbr, wenc_ref[...],
                         preferred_element_type=jnp.float32) + benc_ref[...])
    h3 = h.reshape(GB, K, D)
    nrep = jnp.sum(probs[:, :, None] * h3, axis=1)
    hh = (jnp.dot(t, w1_ref[0:D, :], preferred_element_type=jnp.float32)
          + jnp.dot(nrep, w1_ref[D:2 * D, :],
                    preferred_element_type=jnp.float32) + b1_ref[...])
    g = jnp.dot(hh, w2_ref[...], preferred_element_type=jnp.float32) \
        + b2_ref[...]
    p = jax.nn.sigmoid(g)
    out_ref[...] = jnp.log(p * kprob + (1.0 - p) * mprob_ref[...] + 1e-12)


def _combine(kk3, nbr2d, labels, W_enc, b_enc, text_rep, model_prob,
             W1, b1, W2, b2):
    return pl.pallas_call(
        _comb_body,
        grid=(NG,),
        in_specs=[
            pl.BlockSpec((GB, K, D), lambda g: (g, 0, 0)),
            pl.BlockSpec((GB * K, D), lambda g: (g, 0)),
            pl.BlockSpec((GB, K), lambda g: (g, 0)),
            pl.BlockSpec((D, D), lambda g: (0, 0)),
            pl.BlockSpec((D,), lambda g: (0,)),
            pl.BlockSpec((GB, D), lambda g: (g, 0)),
            pl.BlockSpec((GB, NUM_CLASSES), lambda g: (g, 0)),
            pl.BlockSpec((2 * D, 2 * D), lambda g: (0, 0)),
            pl.BlockSpec((2 * D,), lambda g: (0,)),
            pl.BlockSpec((2 * D, 1), lambda g: (0, 0)),
            pl.BlockSpec((1,), lambda g: (0,)),
        ],
        out_specs=pl.BlockSpec((GB, NUM_CLASSES), lambda g: (g, 0)),
        out_shape=jax.ShapeDtypeStruct((B, NUM_CLASSES), jnp.float32),
    )(kk3, nbr2d, labels, W_enc, b_enc, text_rep, model_prob,
      W1, b1, W2, b2)


def kernel(x, x_mask, x_idx, keys_store, store_vals, neighbor_feats,
           W_enc, b_enc, W_cls, b_cls, W1, b1, W2, b2):
    x_idx = x_idx.astype(jnp.int32)
    text_rep, model_prob = _encode(x, x_mask, W_enc, b_enc, W_cls, b_cls)

    q = _sc_qgather()(x_idx, keys_store)
    scores, bm3 = _scores(q, x_idx, keys_store)
    bm2 = bm3.transpose(1, 0, 2).reshape(B, NBLK)
    tau = _tau(bm2)
    scores2d = scores.reshape(B * NBLK, BLK)
    kk3, nbr3, labels = _sc_select()(scores2d, bm2, tau, keys_store,
                                     neighbor_feats, store_vals)

    return _combine(kk3, nbr3.reshape(B * K, D), labels, W_enc, b_enc,
                    text_rep, model_prob, W1, b1, W2, b2)
